# baseline (speedup 1.0000x reference)
import functools

import jax
import jax.numpy as jnp
from jax.experimental import pallas as pl
from jax.experimental.pallas import tpu as pltpu


def _moe_block_kernel(feat_ref, wr_ref, br_ref, we_ref, be_ref,
                      out_ref, psum_ref, loss_ref, *, n_tokens, n_exp, bal):
    i = pl.program_id(0)
    nsteps = pl.num_programs(0)

    feat = feat_ref[...]
    logits = jnp.dot(feat, wr_ref[...], preferred_element_type=jnp.float32)
    logits = logits + br_ref[...]
    m = jnp.max(logits, axis=1, keepdims=True)
    ex = jnp.exp(logits - m)
    prob = ex / jnp.sum(ex, axis=1, keepdims=True)

    eidx = jax.lax.broadcasted_iota(jnp.int32, prob.shape, 1)
    i1 = jnp.argmax(prob, axis=1).astype(jnp.int32)
    v1 = jnp.max(prob, axis=1)
    masked = jnp.where(eidx == i1[:, None], -1.0, prob)
    i2 = jnp.argmax(masked, axis=1).astype(jnp.int32)
    v2 = jnp.max(masked, axis=1)
    denom = jnp.maximum(v1 + v2, 1e-9)
    w1 = v1 / denom
    w2 = v2 / denom
    w8 = (jnp.where(eidx == i1[:, None], w1[:, None], 0.0)
          + jnp.where(eidx == i2[:, None], w2[:, None], 0.0))

    acc = jnp.dot(w8, be_ref[...], preferred_element_type=jnp.float32)
    for e in range(n_exp):
        neo = jnp.dot(feat, we_ref[e], preferred_element_type=jnp.float32)
        acc = acc + w8[:, e:e + 1] * neo
    out_ref[...] = acc

    block_psum = jnp.sum(prob, axis=0, keepdims=True)
    @pl.when(i == 0)
    def _():
        psum_ref[...] = block_psum
        loss_ref[...] = jnp.zeros_like(loss_ref)

    @pl.when(i > 0)
    def _():
        psum_ref[...] = psum_ref[...] + block_psum

    @pl.when(i == nsteps - 1)
    def _():
        pi = psum_ref[...] / float(n_tokens)
        ent = jnp.sum(pi * jnp.log(jnp.maximum(pi, 1e-9)),
                      axis=1, keepdims=True)
        loss_ref[...] = bal * (ent + jnp.log(float(n_exp)))


def kernel(features, Wr, br, We, be):
    n, d = features.shape
    e, _, o = We.shape
    bm = min(512, n)
    grid = (n // bm,)

    feat_bf = features.astype(jnp.bfloat16)
    wr_bf = Wr.astype(jnp.bfloat16)
    we_bf = We.astype(jnp.bfloat16)

    out, _, loss = pl.pallas_call(
        functools.partial(_moe_block_kernel, n_tokens=n, n_exp=e, bal=0.01),
        grid=grid,
        in_specs=[
            pl.BlockSpec((bm, d), lambda i: (i, 0)),
            pl.BlockSpec((d, e), lambda i: (0, 0)),
            pl.BlockSpec((1, e), lambda i: (0, 0)),
            pl.BlockSpec((e, d, o), lambda i: (0, 0, 0)),
            pl.BlockSpec((e, o), lambda i: (0, 0)),
        ],
        out_specs=[
            pl.BlockSpec((bm, o), lambda i: (i, 0)),
            pl.BlockSpec((1, e), lambda i: (0, 0)),
            pl.BlockSpec((1, 1), lambda i: (0, 0)),
        ],
        out_shape=[
            jax.ShapeDtypeStruct((n, o), jnp.float32),
            jax.ShapeDtypeStruct((1, e), jnp.float32),
            jax.ShapeDtypeStruct((1, 1), jnp.float32),
        ],
    )(feat_bf, wr_bf, br.reshape(1, e), we_bf, be)
    return out, loss.reshape(())


# speedup vs baseline: 1.1657x; 1.1657x over previous
import functools

import jax
import jax.numpy as jnp
from jax.experimental import pallas as pl
from jax.experimental.pallas import tpu as pltpu


def _moe_block_kernel(feat_ref, wr_ref, br_ref, we_ref, be_ref,
                      out_ref, psum_ref, loss_ref, webf_ref,
                      *, n_tokens, n_exp, bal):
    i = pl.program_id(0)
    nsteps = pl.num_programs(0)

    @pl.when(i == 0)
    def _():
        webf_ref[...] = we_ref[...].astype(jnp.bfloat16)

    feat = feat_ref[...].astype(jnp.bfloat16)
    logits = jnp.dot(feat, wr_ref[...], preferred_element_type=jnp.float32)
    logits = logits + br_ref[...]
    m = jnp.max(logits, axis=1, keepdims=True)
    ex = jnp.exp(logits - m)
    prob = ex / jnp.sum(ex, axis=1, keepdims=True)

    eidx = jax.lax.broadcasted_iota(jnp.int32, prob.shape, 1)
    i1 = jnp.argmax(prob, axis=1).astype(jnp.int32)
    v1 = jnp.max(prob, axis=1)
    masked = jnp.where(eidx == i1[:, None], -1.0, prob)
    i2 = jnp.argmax(masked, axis=1).astype(jnp.int32)
    v2 = jnp.max(masked, axis=1)
    denom = jnp.maximum(v1 + v2, 1e-9)
    w1 = v1 / denom
    w2 = v2 / denom
    w8 = (jnp.where(eidx == i1[:, None], w1[:, None], 0.0)
          + jnp.where(eidx == i2[:, None], w2[:, None], 0.0))

    acc = jnp.dot(w8, be_ref[...], preferred_element_type=jnp.float32)
    for e in range(n_exp):
        neo = jnp.dot(feat, webf_ref[e], preferred_element_type=jnp.float32)
        acc = acc + w8[:, e:e + 1] * neo
    out_ref[...] = acc

    block_psum = jnp.sum(prob, axis=0, keepdims=True)
    @pl.when(i == 0)
    def _():
        psum_ref[...] = block_psum
        loss_ref[...] = jnp.zeros_like(loss_ref)

    @pl.when(i > 0)
    def _():
        psum_ref[...] = psum_ref[...] + block_psum

    @pl.when(i == nsteps - 1)
    def _():
        pi = psum_ref[...] / float(n_tokens)
        ent = jnp.sum(pi * jnp.log(jnp.maximum(pi, 1e-9)),
                      axis=1, keepdims=True)
        loss_ref[...] = bal * (ent + jnp.log(float(n_exp)))


def kernel(features, Wr, br, We, be):
    n, d = features.shape
    e, _, o = We.shape
    bm = min(512, n)
    grid = (n // bm,)

    wr_bf = Wr.astype(jnp.bfloat16)

    out, _, loss = pl.pallas_call(
        functools.partial(_moe_block_kernel, n_tokens=n, n_exp=e, bal=0.01),
        grid=grid,
        in_specs=[
            pl.BlockSpec((bm, d), lambda i: (i, 0)),
            pl.BlockSpec((d, e), lambda i: (0, 0)),
            pl.BlockSpec((1, e), lambda i: (0, 0)),
            pl.BlockSpec((e, d, o), lambda i: (0, 0, 0)),
            pl.BlockSpec((e, o), lambda i: (0, 0)),
        ],
        out_specs=[
            pl.BlockSpec((bm, o), lambda i: (i, 0)),
            pl.BlockSpec((1, e), lambda i: (0, 0)),
            pl.BlockSpec((1, 1), lambda i: (0, 0)),
        ],
        out_shape=[
            jax.ShapeDtypeStruct((n, o), jnp.float32),
            jax.ShapeDtypeStruct((1, e), jnp.float32),
            jax.ShapeDtypeStruct((1, 1), jnp.float32),
        ],
        scratch_shapes=[pltpu.VMEM((e, d, o), jnp.bfloat16)],
    )(features, wr_bf, br.reshape(1, e), We, be)
    return out, loss.reshape(())


# speedup vs baseline: 1.2176x; 1.0446x over previous
import functools

import jax
import jax.numpy as jnp
from jax.experimental import pallas as pl
from jax.experimental.pallas import tpu as pltpu


def _moe_block_kernel(feat_ref, wr_ref, br_ref, we_ref, be_ref,
                      out_ref, psum_ref, loss_ref, webf_ref,
                      *, n_tokens, n_exp, bal):
    i = pl.program_id(0)
    nsteps = pl.num_programs(0)

    @pl.when(i == 0)
    def _():
        webf_ref[...] = we_ref[...].astype(jnp.bfloat16)

    feat = feat_ref[...].astype(jnp.bfloat16)
    logits = jnp.dot(feat, wr_ref[...].astype(jnp.bfloat16),
                     preferred_element_type=jnp.float32)
    logits = logits + br_ref[...]
    m = jnp.max(logits, axis=1, keepdims=True)
    ex = jnp.exp(logits - m)
    prob = ex / jnp.sum(ex, axis=1, keepdims=True)

    eidx = jax.lax.broadcasted_iota(jnp.int32, prob.shape, 1)
    i1 = jnp.argmax(prob, axis=1).astype(jnp.int32)
    v1 = jnp.max(prob, axis=1)
    masked = jnp.where(eidx == i1[:, None], -1.0, prob)
    i2 = jnp.argmax(masked, axis=1).astype(jnp.int32)
    v2 = jnp.max(masked, axis=1)
    denom = jnp.maximum(v1 + v2, 1e-9)
    w1 = v1 / denom
    w2 = v2 / denom
    w8 = (jnp.where(eidx == i1[:, None], w1[:, None], 0.0)
          + jnp.where(eidx == i2[:, None], w2[:, None], 0.0))

    acc = jnp.dot(w8, be_ref[...], preferred_element_type=jnp.float32)
    for e in range(n_exp):
        neo = jnp.dot(feat, webf_ref[e], preferred_element_type=jnp.float32)
        acc = acc + w8[:, e:e + 1] * neo
    out_ref[...] = acc

    block_psum = jnp.sum(prob, axis=0, keepdims=True)
    @pl.when(i == 0)
    def _():
        psum_ref[...] = block_psum
        loss_ref[...] = jnp.zeros_like(loss_ref)

    @pl.when(i > 0)
    def _():
        psum_ref[...] = psum_ref[...] + block_psum

    @pl.when(i == nsteps - 1)
    def _():
        pi = psum_ref[...] / float(n_tokens)
        ent = jnp.sum(pi * jnp.log(jnp.maximum(pi, 1e-9)),
                      axis=1, keepdims=True)
        loss_ref[...] = bal * (ent + jnp.log(float(n_exp)))


def kernel(features, Wr, br, We, be):
    n, d = features.shape
    e, _, o = We.shape
    bm = min(1024, n)
    grid = (n // bm,)

    out, _, loss = pl.pallas_call(
        functools.partial(_moe_block_kernel, n_tokens=n, n_exp=e, bal=0.01),
        grid=grid,
        in_specs=[
            pl.BlockSpec((bm, d), lambda i: (i, 0)),
            pl.BlockSpec((d, e), lambda i: (0, 0)),
            pl.BlockSpec((1, e), lambda i: (0, 0)),
            pl.BlockSpec((e, d, o), lambda i: (0, 0, 0)),
            pl.BlockSpec((e, o), lambda i: (0, 0)),
        ],
        out_specs=[
            pl.BlockSpec((bm, o), lambda i: (i, 0)),
            pl.BlockSpec((1, e), lambda i: (0, 0)),
            pl.BlockSpec((1, 1), lambda i: (0, 0)),
        ],
        out_shape=[
            jax.ShapeDtypeStruct((n, o), jnp.float32),
            jax.ShapeDtypeStruct((1, e), jnp.float32),
            jax.ShapeDtypeStruct((1, 1), jnp.float32),
        ],
        scratch_shapes=[pltpu.VMEM((e, d, o), jnp.bfloat16)],
    )(features, Wr, br.reshape(1, e), We, be)
    return out, loss.reshape(())
